# Initial kernel scaffold; baseline (speedup 1.0000x reference)
import jax
import jax.numpy as jnp
from jax.experimental import pallas as pl


def kernel(x, edge_index, edge_attr, rel_w1, rel_b1, rel_w2, rel_b2, msg_w1, msg_b1, msg_w2, msg_b2, att_w1, att_b1, att_w2, att_b2, upd_w1, upd_b1, upd_w2, upd_b2, slf_w1, slf_b1, slf_w2, slf_b2, ln_g, ln_b):
    raise NotImplementedError("write your pallas kernel here")


# speedup vs baseline: 2.3847x; 2.3847x over previous
import functools

import jax
import jax.numpy as jnp
from jax import lax
from jax.experimental import pallas as pl
from jax.experimental.pallas import tpu as pltpu
from jax.experimental.pallas import tpu_sc as plsc

_NC = 2
_NS = 16
_NW = _NC * _NS

_C = 80
_NBUF = 5


def _build_gather(E, N, D):
    EPW = E // _NW
    NCH = EPW // _C
    assert EPW * _NW == E and NCH * _C == EPW and NCH % _NBUF == 0
    mesh = plsc.VectorSubcoreMesh(core_axis_name="c", subcore_axis_name="s")

    @functools.partial(
        pl.kernel,
        out_type=(jax.ShapeDtypeStruct((E, D), jnp.float32),
                  jax.ShapeDtypeStruct((E, D), jnp.float32)),
        mesh=mesh,
        scratch_types=(
            [pltpu.VMEM((EPW,), jnp.int32),
             pltpu.VMEM((EPW,), jnp.int32),
             pltpu.VMEM((_NBUF, _C, D), jnp.float32),
             pltpu.VMEM((_NBUF, _C, D), jnp.float32)]
            + [pltpu.SemaphoreType.DMA] * (2 * _NBUF)
        ),
        compiler_params=pltpu.CompilerParams(needs_layout_passes=False),
    )
    def gather_k(src_h, dst_h, x_h, xs_out, xd_out, sidx, didx, bufs, bufd,
                 *sems):
        gsem = sems[:_NBUF]
        wsem = sems[_NBUF:]
        cid = lax.axis_index("c")
        sid = lax.axis_index("s")
        wid = sid * _NC + cid
        base = wid * EPW
        pltpu.sync_copy(src_h.at[pl.ds(base, EPW)], sidx)
        pltpu.sync_copy(dst_h.at[pl.ds(base, EPW)], didx)

        def issue_gather(j, b):
            pltpu.async_copy(x_h.at[sidx.at[pl.ds(j * _C, _C)]],
                             bufs.at[b], gsem[b])
            pltpu.async_copy(x_h.at[didx.at[pl.ds(j * _C, _C)]],
                             bufd.at[b], gsem[b])

        for b in range(_NBUF):
            issue_gather(b, b)

        @pl.loop(0, NCH, step=_NBUF)
        def _(j0):
            for b in range(_NBUF):
                j = j0 + b
                pltpu.make_async_copy(x_h.at[pl.ds(0, _C)], bufs.at[b],
                                      gsem[b]).wait()
                pltpu.make_async_copy(x_h.at[pl.ds(0, _C)], bufd.at[b],
                                      gsem[b]).wait()
                pltpu.async_copy(bufs.at[b], xs_out.at[pl.ds(base + j * _C, _C)],
                                 wsem[b])
                pltpu.async_copy(bufd.at[b], xd_out.at[pl.ds(base + j * _C, _C)],
                                 wsem[b])

                @pl.when(j + _NBUF < NCH)
                def _():
                    pltpu.make_async_copy(bufs.at[b],
                                          xs_out.at[pl.ds(base, _C)],
                                          wsem[b]).wait()
                    pltpu.make_async_copy(bufd.at[b],
                                          xd_out.at[pl.ds(base, _C)],
                                          wsem[b]).wait()
                    issue_gather(j + _NBUF, b)

        for b in range(_NBUF):
            pltpu.make_async_copy(bufs.at[b], xs_out.at[pl.ds(base, _C)],
                                  wsem[b]).wait()
            pltpu.make_async_copy(bufd.at[b], xd_out.at[pl.ds(base, _C)],
                                  wsem[b]).wait()

    return gather_k


def _build_scatter(E, N, D):
    NH = N // 2
    NHA = NH + 120
    SPAN = NHA // _NS
    EPT = E // _NS
    NCH = EPT // _C
    WSTRIDE = 312
    WSPAN = 320
    assert WSTRIDE * (_NS - 1) + WSPAN == NH and NHA % _NS == 0
    mesh = plsc.VectorSubcoreMesh(core_axis_name="c", subcore_axis_name="s")

    @functools.partial(
        pl.kernel,
        out_type=(jax.ShapeDtypeStruct((_NC, NH, D), jnp.float32),
                  jax.ShapeDtypeStruct((_NS * N,), jnp.float32)),
        mesh=mesh,
        scratch_types=(
            [pltpu.VMEM((EPT,), jnp.int32),
             pltpu.VMEM((1, _C), jnp.int32),
             pltpu.VMEM((_C, D), jnp.float32),
             pltpu.VMEM((_C,), jnp.float32),
             pltpu.VMEM((N,), jnp.float32),
             pltpu.VMEM_SHARED((NHA, D), jnp.float32)]
        ),
        compiler_params=pltpu.CompilerParams(needs_layout_passes=False),
    )
    def scatter_k(dst_h, w_h, rel_h, zeros_h, agg_out, cnt_out,
                  didx1, jadj, wbuf, relbuf, counts_v, agg_sp):
        cid = lax.axis_index("c")
        sid = lax.axis_index("s")
        base = sid * EPT
        lo = cid * NH

        pltpu.sync_copy(dst_h.at[pl.ds(base, EPT)], didx1)

        @pl.when(cid == 0)
        def _():
            @pl.loop(0, N // 16)
            def _(i):
                counts_v[pl.ds(i * 16, 16)] = jnp.zeros((16,), jnp.float32)

        pltpu.sync_copy(zeros_h.at[pl.ds(sid * SPAN, SPAN)],
                        agg_sp.at[pl.ds(sid * SPAN, SPAN)])
        plsc.subcore_barrier()

        @pl.loop(0, NCH)
        def _(j):
            @pl.when(cid == 0)
            def _():
                pltpu.sync_copy(rel_h.at[pl.ds(base + j * _C, _C)], relbuf)
                for k in range(_C // 16):
                    iv = didx1[pl.ds(j * _C + k * 16, 16)]
                    v = relbuf[pl.ds(k * 16, 16)]
                    plsc.addupdate_scatter(counts_v, [iv], v)

            @pl.loop(0, _C // 16)
            def _(k):
                iv = didx1[pl.ds(j * _C + k * 16, 16)]
                shifted = iv - lo
                ok = (shifted >= 0) & (shifted < NH)
                jadj[0, pl.ds(k * 16, 16)] = jnp.where(ok, shifted, NH)

            pltpu.sync_copy(w_h.at[pl.ds(base + j * _C, _C)], wbuf)
            pltpu.sync_copy(wbuf, agg_sp.at[jadj.at[0]], add=True)

        plsc.subcore_barrier()
        pltpu.sync_copy(agg_sp.at[pl.ds(sid * WSTRIDE, WSPAN)],
                        agg_out.at[cid, pl.ds(sid * WSTRIDE, WSPAN)])

        @pl.when(cid == 0)
        def _():
            pltpu.sync_copy(counts_v, cnt_out.at[pl.ds(sid * N, N)])

    return scatter_k


_BE = 512


def _edge_mlp_body(xs_ref, xd_ref, ea_ref, wsrc_ref, wdst_ref, wea_ref,
                   bcat_ref, rw2_ref, rb2_ref, mw2_ref, mb2_ref,
                   w_out_ref, rel_out_ref):
    h = (jnp.dot(xs_ref[...], wsrc_ref[...],
                 preferred_element_type=jnp.float32)
         + jnp.dot(xd_ref[...], wdst_ref[...],
                   preferred_element_type=jnp.float32)
         + jnp.dot(ea_ref[...], wea_ref[...],
                   preferred_element_type=jnp.float32)
         + bcat_ref[...])
    h = jnp.maximum(h, 0.0)
    d = mw2_ref.shape[0]
    rel_logit = jnp.dot(h[:, :d], rw2_ref[...],
                        preferred_element_type=jnp.float32) + rb2_ref[...]
    rel = jax.nn.sigmoid(rel_logit[:, 0])
    msg = jnp.dot(h[:, d:], mw2_ref[...],
                  preferred_element_type=jnp.float32) + mb2_ref[...]
    w_out_ref[...] = msg * (0.25 * rel)[:, None]
    rel_out_ref[...] = rel


def _edge_mlp(xs, xd, ea, wsrc, wdst, wea, bcat, rw2, rb2, mw2, mb2):
    E, D = xs.shape
    ED = ea.shape[1]
    H2 = wsrc.shape[1]
    grid = (E // _BE,)
    fixed = lambda *s: pl.BlockSpec(s, lambda i: (0,) * len(s))
    return pl.pallas_call(
        _edge_mlp_body,
        grid=grid,
        in_specs=[
            pl.BlockSpec((_BE, D), lambda i: (i, 0)),
            pl.BlockSpec((_BE, D), lambda i: (i, 0)),
            pl.BlockSpec((_BE, ED), lambda i: (i, 0)),
            fixed(D, H2), fixed(D, H2), fixed(ED, H2), fixed(1, H2),
            fixed(D, 1), fixed(1, 1), fixed(D, D), fixed(1, D),
        ],
        out_specs=[
            pl.BlockSpec((_BE, D), lambda i: (i, 0)),
            pl.BlockSpec((_BE,), lambda i: (i,)),
        ],
        out_shape=(jax.ShapeDtypeStruct((E, D), jnp.float32),
                   jax.ShapeDtypeStruct((E,), jnp.float32)),
    )(xs, xd, ea, wsrc, wdst, wea, bcat, rw2, rb2, mw2, mb2)


_BN = 1000


def _node_body(x_ref, agg_ref, cnt_ref, sw1_ref, sb1_ref, sw2_ref, sb2_ref,
               uw1a_ref, uw1b_ref, ub1_ref, uw2_ref, ub2_ref,
               g_ref, b_ref, out_ref):
    x = x_ref[...]
    agg = agg_ref[...]
    counts = jnp.sum(cnt_ref[...], axis=1)
    sh = jnp.maximum(jnp.dot(x, sw1_ref[...],
                             preferred_element_type=jnp.float32)
                     + sb1_ref[...], 0.0)
    self_info = jnp.dot(sh, sw2_ref[...],
                        preferred_element_type=jnp.float32) + sb2_ref[...]
    uh = jnp.maximum(jnp.dot(x, uw1a_ref[...],
                             preferred_element_type=jnp.float32)
                     + jnp.dot(agg, uw1b_ref[...],
                               preferred_element_type=jnp.float32)
                     + ub1_ref[...], 0.0)
    update = jnp.dot(uh, uw2_ref[...],
                     preferred_element_type=jnp.float32) + ub2_ref[...]
    iso = jnp.exp(-counts)[:, None]
    y = x + (1.0 - iso) * update + iso * self_info
    mu = jnp.mean(y, axis=-1, keepdims=True)
    var = jnp.mean((y - mu) * (y - mu), axis=-1, keepdims=True)
    out_ref[...] = (y - mu) / jnp.sqrt(var + 1e-5) * g_ref[...] + b_ref[...]


def _node_stage(x, agg2, cnt, sw1, sb1, sw2, sb2, uw1a, uw1b, ub1, uw2, ub2,
                ln_g, ln_b):
    N, D = x.shape
    HID = sw1.shape[1]
    grid = (N // _BN,)
    fixed = lambda *s: pl.BlockSpec(s, lambda i: (0,) * len(s))
    return pl.pallas_call(
        _node_body,
        grid=grid,
        in_specs=[
            pl.BlockSpec((_BN, D), lambda i: (i, 0)),
            pl.BlockSpec((_BN, D), lambda i: (i, 0)),
            pl.BlockSpec((_BN, _NS), lambda i: (i, 0)),
            fixed(D, HID), fixed(1, HID), fixed(HID, D), fixed(1, D),
            fixed(D, HID), fixed(D, HID), fixed(1, HID),
            fixed(HID, D), fixed(1, D),
            fixed(1, D), fixed(1, D),
        ],
        out_specs=pl.BlockSpec((_BN, D), lambda i: (i, 0)),
        out_shape=jax.ShapeDtypeStruct((N, D), jnp.float32),
    )(x, agg2, cnt, sw1, sb1, sw2, sb2, uw1a, uw1b, ub1, uw2, ub2, ln_g, ln_b)


def kernel(x, edge_index, edge_attr,
           rel_w1, rel_b1, rel_w2, rel_b2,
           msg_w1, msg_b1, msg_w2, msg_b2,
           att_w1, att_b1, att_w2, att_b2,
           upd_w1, upd_b1, upd_w2, upd_b2,
           slf_w1, slf_b1, slf_w2, slf_b2,
           ln_g, ln_b):
    N, D = x.shape
    E = edge_index.shape[1]
    EPW = E // _NW
    NCH = EPW // _C

    src = edge_index[0]
    dst = edge_index[1]

    xs, xd = _build_gather(E, N, D)(src, dst, x)

    wsrc = jnp.concatenate([rel_w1[:D], msg_w1[D:2 * D]], axis=1)
    wdst = jnp.concatenate([rel_w1[D:2 * D], msg_w1[:D]], axis=1)
    wea = jnp.concatenate([rel_w1[2 * D:], msg_w1[2 * D:]], axis=1)
    bcat = jnp.concatenate([rel_b1, msg_b1])[None, :]

    weighted, reliability = _edge_mlp(
        xs, xd, edge_attr, wsrc, wdst, wea, bcat,
        rel_w2, rel_b2[None, :], msg_w2, msg_b2[None, :])

    zeros = jnp.zeros((N // 2 + 120, D), jnp.float32)
    agg2, cnt = _build_scatter(E, N, D)(dst, weighted, reliability, zeros)

    out = _node_stage(
        x, agg2.reshape(N, D), cnt.reshape(_NS, N).T,
        slf_w1, slf_b1[None, :], slf_w2, slf_b2[None, :],
        upd_w1[:D], upd_w1[D:], upd_b1[None, :], upd_w2, upd_b2[None, :],
        ln_g[None, :], ln_b[None, :])

    return (out, reliability)


# speedup vs baseline: 2.8059x; 1.1766x over previous
import functools

import jax
import jax.numpy as jnp
from jax import lax
from jax.experimental import pallas as pl
from jax.experimental.pallas import tpu as pltpu
from jax.experimental.pallas import tpu_sc as plsc

_NC = 2
_NS = 16
_NW = _NC * _NS

_C = 80
_NBUF = 5


def _build_gather(E, N, D):
    EPW = E // _NW
    NCH = EPW // _C
    assert EPW * _NW == E and NCH * _C == EPW and NCH % _NBUF == 0
    mesh = plsc.VectorSubcoreMesh(core_axis_name="c", subcore_axis_name="s")

    @functools.partial(
        pl.kernel,
        out_type=(jax.ShapeDtypeStruct((E, D), jnp.float32),
                  jax.ShapeDtypeStruct((E, D), jnp.float32)),
        mesh=mesh,
        scratch_types=(
            [pltpu.VMEM((EPW,), jnp.int32),
             pltpu.VMEM((EPW,), jnp.int32),
             pltpu.VMEM((_NBUF, _C, D), jnp.float32),
             pltpu.VMEM((_NBUF, _C, D), jnp.float32)]
            + [pltpu.SemaphoreType.DMA] * (2 * _NBUF)
        ),
        compiler_params=pltpu.CompilerParams(needs_layout_passes=False),
    )
    def gather_k(src_h, dst_h, x_h, xs_out, xd_out, sidx, didx, bufs, bufd,
                 *sems):
        gsem = sems[:_NBUF]
        wsem = sems[_NBUF:]
        cid = lax.axis_index("c")
        sid = lax.axis_index("s")
        wid = sid * _NC + cid
        base = wid * EPW
        pltpu.sync_copy(src_h.at[pl.ds(base, EPW)], sidx)
        pltpu.sync_copy(dst_h.at[pl.ds(base, EPW)], didx)

        def issue_gather(j, b):
            pltpu.async_copy(x_h.at[sidx.at[pl.ds(j * _C, _C)]],
                             bufs.at[b], gsem[b])
            pltpu.async_copy(x_h.at[didx.at[pl.ds(j * _C, _C)]],
                             bufd.at[b], gsem[b])

        for b in range(_NBUF):
            issue_gather(b, b)

        @pl.loop(0, NCH, step=_NBUF)
        def _(j0):
            for b in range(_NBUF):
                j = j0 + b
                pltpu.make_async_copy(x_h.at[pl.ds(0, _C)], bufs.at[b],
                                      gsem[b]).wait()
                pltpu.make_async_copy(x_h.at[pl.ds(0, _C)], bufd.at[b],
                                      gsem[b]).wait()
                pltpu.async_copy(bufs.at[b], xs_out.at[pl.ds(base + j * _C, _C)],
                                 wsem[b])
                pltpu.async_copy(bufd.at[b], xd_out.at[pl.ds(base + j * _C, _C)],
                                 wsem[b])

                @pl.when(j + _NBUF < NCH)
                def _():
                    pltpu.make_async_copy(bufs.at[b],
                                          xs_out.at[pl.ds(base, _C)],
                                          wsem[b]).wait()
                    pltpu.make_async_copy(bufd.at[b],
                                          xd_out.at[pl.ds(base, _C)],
                                          wsem[b]).wait()
                    issue_gather(j + _NBUF, b)

        for b in range(_NBUF):
            pltpu.make_async_copy(bufs.at[b], xs_out.at[pl.ds(base, _C)],
                                  wsem[b]).wait()
            pltpu.make_async_copy(bufd.at[b], xd_out.at[pl.ds(base, _C)],
                                  wsem[b]).wait()

    return gather_k


def _build_scatter(E, N, D):
    NH = N // 2
    NHA = NH + 8
    SPAN = 320
    EPT = E // _NS
    NCH = EPT // _C
    NB2 = 2
    WSTRIDE = 312
    WSPAN = 320
    assert WSTRIDE * (_NS - 1) + WSPAN == NH and NHA % _NS == 0
    mesh = plsc.VectorSubcoreMesh(core_axis_name="c", subcore_axis_name="s")

    @functools.partial(
        pl.kernel,
        out_type=(jax.ShapeDtypeStruct((_NC, NH, D), jnp.float32),
                  jax.ShapeDtypeStruct((_NW * N,), jnp.float32)),
        mesh=mesh,
        scratch_types=(
            [pltpu.VMEM((EPT,), jnp.int32),
             pltpu.VMEM((1, _C), jnp.int32),
             pltpu.VMEM((NB2, _C), jnp.int32),
             pltpu.VMEM((NB2, _C, D), jnp.float32),
             pltpu.VMEM((_C,), jnp.float32),
             pltpu.VMEM((N,), jnp.float32),
             pltpu.VMEM_SHARED((NHA, D), jnp.float32)]
            + [pltpu.SemaphoreType.DMA] * NB2
        ),
        compiler_params=pltpu.CompilerParams(needs_layout_passes=False),
    )
    def scatter_k(dst_h, w_h, rel_h, zeros_h, agg_out, cnt_out,
                  didx1, jadj, ibuf, wbuf, relbuf, counts_v, agg_sp, *gsem):
        cid = lax.axis_index("c")
        sid = lax.axis_index("s")
        base = sid * EPT
        lo = cid * NH
        cnt_lo = cid * (NCH // 2)
        cnt_hi = cnt_lo + NCH // 2

        pltpu.sync_copy(dst_h.at[pl.ds(base, EPT)], didx1)

        @pl.loop(0, N // 16)
        def _(i):
            counts_v[pl.ds(i * 16, 16)] = jnp.zeros((16,), jnp.float32)

        @pl.when(sid < _NS - 1)
        def _():
            pltpu.sync_copy(zeros_h.at[pl.ds(sid * SPAN, SPAN)],
                            agg_sp.at[pl.ds(sid * SPAN, SPAN)])

        @pl.when(sid == _NS - 1)
        def _():
            pltpu.sync_copy(zeros_h.at[pl.ds((_NS - 1) * SPAN, NHA - (_NS - 1) * SPAN)],
                            agg_sp.at[pl.ds((_NS - 1) * SPAN, NHA - (_NS - 1) * SPAN)])

        plsc.subcore_barrier()

        def issue(j, b):
            for k in range(_C // 16):
                ibuf[b, pl.ds(k * 16, 16)] = (
                    lax.iota(jnp.int32, 16) + (base + j * _C + k * 16))
            pltpu.async_copy(w_h.at[ibuf.at[b]], wbuf.at[b], gsem[b])

        for b in range(NB2):
            issue(b, b)

        @pl.loop(0, NCH, step=NB2)
        def _(j0):
            for b in range(NB2):
                j = j0 + b
                pltpu.make_async_copy(w_h.at[pl.ds(0, _C)], wbuf.at[b],
                                      gsem[b]).wait()

                @pl.when((j >= cnt_lo) & (j < cnt_hi))
                def _():
                    pltpu.sync_copy(rel_h.at[pl.ds(base + j * _C, _C)],
                                    relbuf)
                    for k in range(_C // 16):
                        iv = didx1[pl.ds(j * _C + k * 16, 16)]
                        v = relbuf[pl.ds(k * 16, 16)]
                        plsc.addupdate_scatter(counts_v, [iv], v)

                @pl.loop(0, _C // 16)
                def _(k):
                    iv = didx1[pl.ds(j * _C + k * 16, 16)]
                    shifted = iv - lo
                    ok = (shifted >= 0) & (shifted < NH)
                    jadj[0, pl.ds(k * 16, 16)] = jnp.where(ok, shifted, NH)

                pltpu.sync_copy(wbuf.at[b], agg_sp.at[jadj.at[0]], add=True)

                @pl.when(j + NB2 < NCH)
                def _():
                    issue(j + NB2, b)

        plsc.subcore_barrier()
        pltpu.sync_copy(agg_sp.at[pl.ds(sid * WSTRIDE, WSPAN)],
                        agg_out.at[cid, pl.ds(sid * WSTRIDE, WSPAN)])
        pltpu.sync_copy(counts_v,
                        cnt_out.at[pl.ds((sid * _NC + cid) * N, N)])

    return scatter_k


_BE = 512


def _edge_mlp_body(xs_ref, xd_ref, ea_ref, wsrc_ref, wdst_ref, wea_ref,
                   bcat_ref, rw2_ref, rb2_ref, mw2_ref, mb2_ref,
                   w_out_ref, rel_out_ref):
    h = (jnp.dot(xs_ref[...], wsrc_ref[...],
                 preferred_element_type=jnp.float32)
         + jnp.dot(xd_ref[...], wdst_ref[...],
                   preferred_element_type=jnp.float32)
         + jnp.dot(ea_ref[...], wea_ref[...],
                   preferred_element_type=jnp.float32)
         + bcat_ref[...])
    h = jnp.maximum(h, 0.0)
    d = mw2_ref.shape[0]
    rel_logit = jnp.dot(h[:, :d], rw2_ref[...],
                        preferred_element_type=jnp.float32) + rb2_ref[...]
    rel = jax.nn.sigmoid(rel_logit[:, 0])
    msg = jnp.dot(h[:, d:], mw2_ref[...],
                  preferred_element_type=jnp.float32) + mb2_ref[...]
    w_out_ref[...] = msg * (0.25 * rel)[:, None]
    rel_out_ref[...] = rel


def _edge_mlp(xs, xd, ea, wsrc, wdst, wea, bcat, rw2, rb2, mw2, mb2):
    E, D = xs.shape
    ED = ea.shape[1]
    H2 = wsrc.shape[1]
    grid = (E // _BE,)
    fixed = lambda *s: pl.BlockSpec(s, lambda i: (0,) * len(s))
    return pl.pallas_call(
        _edge_mlp_body,
        grid=grid,
        in_specs=[
            pl.BlockSpec((_BE, D), lambda i: (i, 0)),
            pl.BlockSpec((_BE, D), lambda i: (i, 0)),
            pl.BlockSpec((_BE, ED), lambda i: (i, 0)),
            fixed(D, H2), fixed(D, H2), fixed(ED, H2), fixed(1, H2),
            fixed(D, 1), fixed(1, 1), fixed(D, D), fixed(1, D),
        ],
        out_specs=[
            pl.BlockSpec((_BE, D), lambda i: (i, 0)),
            pl.BlockSpec((_BE,), lambda i: (i,)),
        ],
        out_shape=(jax.ShapeDtypeStruct((E, D), jnp.float32),
                   jax.ShapeDtypeStruct((E,), jnp.float32)),
    )(xs, xd, ea, wsrc, wdst, wea, bcat, rw2, rb2, mw2, mb2)


_BN = 1000


def _node_body(x_ref, agg_ref, cnt_ref, sw1_ref, sb1_ref, sw2_ref, sb2_ref,
               uw1a_ref, uw1b_ref, ub1_ref, uw2_ref, ub2_ref,
               g_ref, b_ref, out_ref):
    x = x_ref[...]
    agg = agg_ref[...]
    counts = jnp.sum(cnt_ref[...], axis=1)
    sh = jnp.maximum(jnp.dot(x, sw1_ref[...],
                             preferred_element_type=jnp.float32)
                     + sb1_ref[...], 0.0)
    self_info = jnp.dot(sh, sw2_ref[...],
                        preferred_element_type=jnp.float32) + sb2_ref[...]
    uh = jnp.maximum(jnp.dot(x, uw1a_ref[...],
                             preferred_element_type=jnp.float32)
                     + jnp.dot(agg, uw1b_ref[...],
                               preferred_element_type=jnp.float32)
                     + ub1_ref[...], 0.0)
    update = jnp.dot(uh, uw2_ref[...],
                     preferred_element_type=jnp.float32) + ub2_ref[...]
    iso = jnp.exp(-counts)[:, None]
    y = x + (1.0 - iso) * update + iso * self_info
    mu = jnp.mean(y, axis=-1, keepdims=True)
    var = jnp.mean((y - mu) * (y - mu), axis=-1, keepdims=True)
    out_ref[...] = (y - mu) / jnp.sqrt(var + 1e-5) * g_ref[...] + b_ref[...]


def _node_stage(x, agg2, cnt, sw1, sb1, sw2, sb2, uw1a, uw1b, ub1, uw2, ub2,
                ln_g, ln_b):
    N, D = x.shape
    HID = sw1.shape[1]
    grid = (N // _BN,)
    fixed = lambda *s: pl.BlockSpec(s, lambda i: (0,) * len(s))
    return pl.pallas_call(
        _node_body,
        grid=grid,
        in_specs=[
            pl.BlockSpec((_BN, D), lambda i: (i, 0)),
            pl.BlockSpec((_BN, D), lambda i: (i, 0)),
            pl.BlockSpec((_BN, _NW), lambda i: (i, 0)),
            fixed(D, HID), fixed(1, HID), fixed(HID, D), fixed(1, D),
            fixed(D, HID), fixed(D, HID), fixed(1, HID),
            fixed(HID, D), fixed(1, D),
            fixed(1, D), fixed(1, D),
        ],
        out_specs=pl.BlockSpec((_BN, D), lambda i: (i, 0)),
        out_shape=jax.ShapeDtypeStruct((N, D), jnp.float32),
    )(x, agg2, cnt, sw1, sb1, sw2, sb2, uw1a, uw1b, ub1, uw2, ub2, ln_g, ln_b)


def kernel(x, edge_index, edge_attr,
           rel_w1, rel_b1, rel_w2, rel_b2,
           msg_w1, msg_b1, msg_w2, msg_b2,
           att_w1, att_b1, att_w2, att_b2,
           upd_w1, upd_b1, upd_w2, upd_b2,
           slf_w1, slf_b1, slf_w2, slf_b2,
           ln_g, ln_b):
    N, D = x.shape
    E = edge_index.shape[1]
    EPW = E // _NW
    NCH = EPW // _C

    src = edge_index[0]
    dst = edge_index[1]

    xs, xd = _build_gather(E, N, D)(src, dst, x)

    wsrc = jnp.concatenate([rel_w1[:D], msg_w1[D:2 * D]], axis=1)
    wdst = jnp.concatenate([rel_w1[D:2 * D], msg_w1[:D]], axis=1)
    wea = jnp.concatenate([rel_w1[2 * D:], msg_w1[2 * D:]], axis=1)
    bcat = jnp.concatenate([rel_b1, msg_b1])[None, :]

    weighted, reliability = _edge_mlp(
        xs, xd, edge_attr, wsrc, wdst, wea, bcat,
        rel_w2, rel_b2[None, :], msg_w2, msg_b2[None, :])

    zeros = jnp.zeros((N // 2 + 8, D), jnp.float32)
    agg2, cnt = _build_scatter(E, N, D)(dst, weighted, reliability, zeros)

    out = _node_stage(
        x, agg2.reshape(N, D), cnt.reshape(_NW, N).T,
        slf_w1, slf_b1[None, :], slf_w2, slf_b2[None, :],
        upd_w1[:D], upd_w1[D:], upd_b1[None, :], upd_w2, upd_b2[None, :],
        ln_g[None, :], ln_b[None, :])

    return (out, reliability)
